# baseline (speedup 1.0000x reference)
import functools

import jax
import jax.numpy as jnp
from jax import lax
from jax.experimental import pallas as pl
from jax.experimental.pallas import tpu as pltpu
from jax.experimental.pallas import tpu_sc as plsc

NC = 2
NS = 16
L = 16
NW = NC * NS

B = 16384
POS = B * 3
D = 64
NE = 1_000_000

CB = 2
NBPW = 244
NCH = NBPW // CB
EPW = NBPW * 128
MAIN = NW * EPW
TAIL = NE - MAIN
ECAP = 8192
RCAP = 4096
CCAP = 1024
SLAB = 3072
MAGIC3 = 43691


def _lshr(x, k):
    u = lax.bitcast_convert_type(x, jnp.uint32)
    return lax.bitcast_convert_type(u >> jnp.uint32(k), jnp.int32)


def _lshl(x, k):
    u = lax.bitcast_convert_type(x, jnp.uint32)
    return lax.bitcast_convert_type(u << jnp.uint32(k), jnp.int32)


def _unpack(hv):
    return _lshr(hv, 17), hv & 0x1FFFF


def _edge_role(pos):
    edge = _lshr(pos * MAGIC3, 17)
    role = pos - edge * 3
    return edge, role


def _sweep_body(edge_ref, entT_ref, relT_ref, etail_ref, rtail_ref, out_ref,
                bufA, bufB, slab, ehits, rhits, clist, stag, sidx, out_v,
                S, sem):
    c = lax.axis_index("c")
    s = lax.axis_index("s")
    wid = s * NC + c
    my_lo = wid * EPW
    lanes = lax.iota(jnp.int32, L)

    def zrow(i, carry):
        for k in range(8):
            bufA[i, pl.ds(k * L, L)] = jnp.zeros((L,), jnp.float32)
        return carry
    lax.fori_loop(0, 128, zrow, 0)
    for z in range(4):
        pltpu.sync_copy(bufA, S.at[pl.ds(s * 512 + z * 128, 128)])
    plsc.subcore_barrier()

    def scan_group(i, carry):
        eoff, roff, p0 = carry
        v = slab[pl.ds(i * L, L)]
        pos = p0 + i * L + lanes
        _, role = _edge_role(pos)
        inr = (v >= my_lo) & (v < my_lo + EPW)
        packed = _lshl(v - my_lo, 17) | pos
        me = inr & (role != 1)
        mr = inr & (role == 1)
        plsc.store_compressed(
            ehits.at[pl.ds(jnp.minimum(eoff, ECAP - L), L)], packed, mask=me)
        plsc.store_compressed(
            rhits.at[pl.ds(jnp.minimum(roff, RCAP - L), L)], packed, mask=mr)
        return (eoff + jnp.sum(me.astype(jnp.int32)),
                roff + jnp.sum(mr.astype(jnp.int32)), p0)

    eoff = jnp.int32(0)
    roff = jnp.int32(0)
    for sl in range(POS // SLAB):
        pltpu.sync_copy(edge_ref.at[pl.ds(sl * SLAB, SLAB)], slab)
        eoff, roff, _ = lax.fori_loop(
            0, SLAB // L, scan_group, (eoff, roff, jnp.int32(sl * SLAB)))

    def fire(tbl, buf, ch):
        start = my_lo + ch * (CB * 128)
        for dt in range(8):
            for blk in range(CB):
                pltpu.async_copy(
                    tbl.at[pl.ds(dt * 8, 8), pl.ds(start + blk * 128, 128)],
                    buf.at[pl.ds((dt * CB + blk) * 8, 8), :], sem)

    def drain(tbl, buf):
        for k in range(8 * CB):
            pltpu.make_async_copy(
                tbl.at[pl.ds(0, 8), pl.ds(0, 128)],
                buf.at[pl.ds(k * 8, 8), :], sem).wait()

    def dense_group(buf, hv, valid, is_rel):
        vloc, pos = _unpack(hv)
        vloc = jnp.where(valid, vloc, 0)
        pos = jnp.where(valid, pos, 0)
        blk = _lshr(vloc, 7)
        il = vloc & 127
        edge, role = _edge_role(pos)
        if is_rel:
            sign = jnp.ones((L,), jnp.float32)
        else:
            sign = jnp.where(role == 2, -1.0, 1.0).astype(jnp.float32)
        srow = _lshr(edge, 1)
        half = (edge & 1) * 64
        rowb = blk * 8
        for d in range(D):
            rowc = (d >> 3) * (CB * 8) + (d & 7)
            val = plsc.load_gather(buf, [rowb + rowc, il])
            cold = half + d
            plsc.store_scatter(stag, [lanes, cold], val * sign)
            plsc.store_scatter(stag, [lanes, cold ^ 64],
                               jnp.zeros((L,), jnp.float32))
        sidx[...] = jnp.where(valid, srow, -1)
        pltpu.sync_copy(stag, S.at[plsc.Indices(sidx, ignored_value=-1)],
                        add=True)

    def process_chunk(ch, buf, hits, hoff, hcap, is_rel):
        win_lo = ch * (CB * 128)

        def cgroup(i, cfill):
            hv = hits[pl.ds(i * L, L)]
            valid = (i * L + lanes) < hoff
            vloc = _lshr(hv, 17)
            inwin = valid & (vloc >= win_lo) & (vloc < win_lo + CB * 128)
            plsc.store_compressed(
                clist.at[pl.ds(jnp.minimum(cfill, CCAP - L), L)], hv, mask=inwin)
            return cfill + jnp.sum(inwin.astype(jnp.int32))

        ng = lax.div(hoff + (L - 1), L)
        cfill = lax.fori_loop(0, ng, cgroup, jnp.int32(0))

        def dgroup(i, carry):
            hv = clist[pl.ds(i * L, L)]
            valid = (i * L + lanes) < cfill
            vloc, pos = _unpack(hv)
            hv_local = _lshl(vloc - win_lo, 17) | pos
            dense_group(buf, hv_local, valid, is_rel)
            return carry

        ncg = lax.div(cfill + (L - 1), L)
        lax.fori_loop(0, ncg, dgroup, 0)

    def sweep(tbl, hits, hoff, hcap, is_rel):
        fire(tbl, bufA, 0)

        def pair(g, carry):
            ch = 2 * g
            fire(tbl, bufB, ch + 1)
            drain(tbl, bufA)
            process_chunk(ch, bufA, hits, hoff, hcap, is_rel)
            fire(tbl, bufA, ch + 2)
            drain(tbl, bufB)
            process_chunk(ch + 1, bufB, hits, hoff, hcap, is_rel)
            return carry

        lax.fori_loop(0, NCH // 2, pair, 0)
        drain(tbl, bufA)

    sweep(entT_ref, ehits, eoff, ECAP, False)
    sweep(relT_ref, rhits, roff, RCAP, True)

    def tail_pass(tail_ref, is_rel):
        pltpu.sync_copy(tail_ref.at[pl.ds(0, 128)], bufA)
        pltpu.sync_copy(tail_ref.at[pl.ds(128, 128)], bufB)
        pltpu.sync_copy(edge_ref.at[pl.ds(wid * (POS // NW), POS // NW)],
                        slab.at[pl.ds(0, POS // NW)])

        def tgroup(i, carry):
            v = slab[pl.ds(i * L, L)]
            pos = wid * (POS // NW) + i * L + lanes
            _, role = _edge_role(pos)
            mrole = (role == 1) if is_rel else (role != 1)
            mt = mrole & (v >= MAIN)
            eloc = jnp.where(mt, v - MAIN, 0)
            pvec = eloc * D

            def do(sub_lo, buf, m):
                @pl.when(jnp.any(m))
                def _():
                    p = jnp.where(m, pvec - sub_lo * D, 0)
                    row0 = _lshr(p, 7)
                    col0 = p & 127
                    posx = jnp.where(m, pos, 0)
                    edge, role2 = _edge_role(posx)
                    if is_rel:
                        sign = jnp.ones((L,), jnp.float32)
                    else:
                        sign = jnp.where(role2 == 2, -1.0, 1.0).astype(jnp.float32)
                    srow = _lshr(edge, 1)
                    half = (edge & 1) * 64
                    for d in range(D):
                        pr = row0 + (d >> 7)
                        pc = col0 + d
                        pr = pr + _lshr(pc, 7)
                        pc = pc & 127
                        val = plsc.load_gather(buf, [pr, pc])
                        cold = half + d
                        plsc.store_scatter(stag, [lanes, cold], val * sign)
                        plsc.store_scatter(stag, [lanes, cold ^ 64],
                                           jnp.zeros((L,), jnp.float32))
                    sidx[...] = jnp.where(m, srow, -1)
                    pltpu.sync_copy(
                        stag, S.at[plsc.Indices(sidx, ignored_value=-1)],
                        add=True)

            do(0, bufA, mt & (eloc < 256))
            do(256, bufB, mt & (eloc >= 256) & (eloc < 512))
            return carry

        lax.fori_loop(0, (POS // NW) // L, tgroup, 0)

        pltpu.sync_copy(tail_ref.at[pl.ds(256, 32)], bufA.at[pl.ds(0, 32)])

        def tgroup2(i, carry):
            v = slab[pl.ds(i * L, L)]
            pos = wid * (POS // NW) + i * L + lanes
            _, role = _edge_role(pos)
            mrole = (role == 1) if is_rel else (role != 1)
            mt = mrole & (v >= MAIN)
            eloc = jnp.where(mt, v - MAIN, 0)
            pvec = eloc * D

            def do(sub_lo, buf, m):
                @pl.when(jnp.any(m))
                def _():
                    p = jnp.where(m, pvec - sub_lo * D, 0)
                    row0 = _lshr(p, 7)
                    col0 = p & 127
                    posx = jnp.where(m, pos, 0)
                    edge, role2 = _edge_role(posx)
                    if is_rel:
                        sign = jnp.ones((L,), jnp.float32)
                    else:
                        sign = jnp.where(role2 == 2, -1.0, 1.0).astype(jnp.float32)
                    srow = _lshr(edge, 1)
                    half = (edge & 1) * 64
                    for d in range(D):
                        pc = col0 + d
                        pr = row0 + _lshr(pc, 7)
                        pc = pc & 127
                        val = plsc.load_gather(buf, [pr, pc])
                        cold = half + d
                        plsc.store_scatter(stag, [lanes, cold], val * sign)
                        plsc.store_scatter(stag, [lanes, cold ^ 64],
                                           jnp.zeros((L,), jnp.float32))
                    sidx[...] = jnp.where(m, srow, -1)
                    pltpu.sync_copy(
                        stag, S.at[plsc.Indices(sidx, ignored_value=-1)],
                        add=True)

            do(512, bufA, mt & (eloc >= 512))
            return carry

        lax.fori_loop(0, (POS // NW) // L, tgroup2, 0)

    tail_pass(etail_ref, False)
    tail_pass(rtail_ref, True)

    plsc.subcore_barrier()
    pltpu.sync_copy(S.at[pl.ds(s * 512, 512)],
                    out_ref.at[c, pl.ds(s * 512, 512), :])


def _merge_body(parts_ref, out_ref, p0, p1, partE, partO, out_v, sem):
    c = lax.axis_index("c")
    s = lax.axis_index("s")
    wid = s * NC + c
    r0 = wid * 256
    lanes = lax.iota(jnp.int32, L)
    pltpu.sync_copy(parts_ref.at[0, pl.ds(r0, 256), :], p0)
    pltpu.sync_copy(parts_ref.at[1, pl.ds(r0, 256), :], p1)

    def group(g, carry):
        for e in range(L):
            row = g * L + e
            accE = None
            accO = None
            for k in range(8):
                sl = pl.ds(k * L, L)
                v = jnp.abs(p0[row, sl] + p1[row, sl])
                if k < 4:
                    accE = v if accE is None else accE + v
                else:
                    accO = v if accO is None else accO + v
            partE[e, :] = accE
            partO[e, :] = accO
        totE = jnp.zeros((L,), jnp.float32)
        totO = jnp.zeros((L,), jnp.float32)
        for j in range(L):
            cj = jnp.full((L,), j, jnp.int32)
            totE = totE + plsc.load_gather(partE, [lanes, cj])
            totO = totO + plsc.load_gather(partO, [lanes, cj])
        base = g * 2 * L
        plsc.store_scatter(out_v, [base + 2 * lanes], totE)
        plsc.store_scatter(out_v, [base + 2 * lanes + 1], totO)
        return carry

    lax.fori_loop(0, 16, group, 0)
    pltpu.sync_copy(out_v, out_ref.at[pl.ds(wid * 512, 512)])


@functools.partial(jax.jit, static_argnames=())
def _transe_sc(edge_flat, entT, relT, etail, rtail):
    mesh = plsc.VectorSubcoreMesh(core_axis_name="c", subcore_axis_name="s")
    cp = pltpu.CompilerParams(
        needs_layout_passes=False, use_tc_tiling_on_sc=True)
    k1 = pl.kernel(
        _sweep_body,
        out_type=jax.ShapeDtypeStruct((2, 8192, 128), jnp.float32),
        mesh=mesh,
        compiler_params=cp,
        scratch_types=[
            pltpu.VMEM((8 * CB * 8, 128), jnp.float32),
            pltpu.VMEM((8 * CB * 8, 128), jnp.float32),
            pltpu.VMEM((SLAB,), jnp.int32),
            pltpu.VMEM((ECAP,), jnp.int32),
            pltpu.VMEM((RCAP,), jnp.int32),
            pltpu.VMEM((CCAP,), jnp.int32),
            pltpu.VMEM((L, 128), jnp.float32),
            pltpu.VMEM((L,), jnp.int32),
            pltpu.VMEM((128,), jnp.float32),
            pltpu.VMEM_SHARED((8192, 128), jnp.float32),
            pltpu.SemaphoreType.DMA,
        ],
    )
    parts = k1(edge_flat, entT, relT, etail, rtail)
    k2 = pl.kernel(
        _merge_body,
        out_type=jax.ShapeDtypeStruct((B,), jnp.float32),
        mesh=mesh,
        compiler_params=cp,
        scratch_types=[
            pltpu.VMEM((256, 128), jnp.float32),
            pltpu.VMEM((256, 128), jnp.float32),
            pltpu.VMEM((L, L), jnp.float32),
            pltpu.VMEM((L, L), jnp.float32),
            pltpu.VMEM((512,), jnp.float32),
            pltpu.SemaphoreType.DMA,
        ],
    )
    return k2(parts)


def kernel(edge, entity_embedding, relation_embedding):
    etail = entity_embedding[MAIN:].reshape(TAIL * D // 128, 128)
    rtail = relation_embedding[MAIN:].reshape(TAIL * D // 128, 128)
    return _transe_sc(edge.reshape(-1), entity_embedding.T,
                      relation_embedding.T, etail, rtail)


# speedup vs baseline: 1.3781x; 1.3781x over previous
import functools

import jax
import jax.numpy as jnp
from jax import lax
from jax.experimental import pallas as pl
from jax.experimental.pallas import tpu as pltpu
from jax.experimental.pallas import tpu_sc as plsc

NC = 2
NS = 16
L = 16
NW = NC * NS

B = 16384
POS = B * 3
D = 64
NE = 1_000_000

CB = 2
NBPW = 244
NCH = NBPW // CB
EPW = NBPW * 128
MAIN = NW * EPW
TAIL = NE - MAIN
ECAP = 6144
RCAP = 3072
CCAP = 1024
SLAB = 1536
MAGIC3 = 43691


def _lshr(x, k):
    u = lax.bitcast_convert_type(x, jnp.uint32)
    return lax.bitcast_convert_type(u >> jnp.uint32(k), jnp.int32)


def _lshl(x, k):
    u = lax.bitcast_convert_type(x, jnp.uint32)
    return lax.bitcast_convert_type(u << jnp.uint32(k), jnp.int32)


def _unpack(hv):
    return _lshr(hv, 17), hv & 0x1FFFF


def _edge_role(pos):
    edge = _lshr(pos * MAGIC3, 17)
    role = pos - edge * 3
    return edge, role


def _sweep_body(edge_ref, entT_ref, relT_ref, etail_ref, rtail_ref, out_ref,
                bufA, bufB, slab, ehits, rhits, ebuck, rbuck, clist,
                stag, sidxa, soff_sm, S, sem):
    c = lax.axis_index("c")
    s = lax.axis_index("s")
    wid = s * NC + c
    my_lo = wid * EPW
    lanes = lax.iota(jnp.int32, L)

    def zrow(i, carry):
        for k in range(8):
            stag[i, pl.ds(k * L, L)] = jnp.zeros((L,), jnp.float32)
        return carry
    lax.fori_loop(0, 64, zrow, 0)
    for z in range(8):
        pltpu.sync_copy(stag, S.at[pl.ds(s * 512 + z * 64, 64)])
    for z in range(4):
        sidxa[pl.ds(z * L, L)] = jnp.full((L,), -1, jnp.int32)
    plsc.subcore_barrier()

    def scan_group(i, carry):
        eoff, roff, p0 = carry
        v = slab[pl.ds(i * L, L)]
        pos = p0 + i * L + lanes
        _, role = _edge_role(pos)
        inr = (v >= my_lo) & (v < my_lo + EPW)
        packed = _lshl(v - my_lo, 17) | pos
        me = inr & (role != 1)
        mr = inr & (role == 1)
        plsc.store_compressed(
            ehits.at[pl.ds(jnp.minimum(eoff, ECAP - L), L)], packed, mask=me)
        plsc.store_compressed(
            rhits.at[pl.ds(jnp.minimum(roff, RCAP - L), L)], packed, mask=mr)
        return (eoff + jnp.sum(me.astype(jnp.int32)),
                roff + jnp.sum(mr.astype(jnp.int32)), p0)

    eoff = jnp.int32(0)
    roff = jnp.int32(0)
    for sl in range(POS // SLAB):
        pltpu.sync_copy(edge_ref.at[pl.ds(sl * SLAB, SLAB)], slab)
        eoff, roff, _ = lax.fori_loop(
            0, SLAB // L, scan_group, (eoff, roff, jnp.int32(sl * SLAB)))

    def bucketize(hits, hoff, hcap, dst, base_sm):
        ng = lax.div(hoff + (L - 1), L)
        fill = jnp.int32(0)
        for o in range(8):
            soff_sm[base_sm + o] = fill

            def bgroup(i, f, _o=o):
                hv = hits[pl.ds(i * L, L)]
                valid = (i * L + lanes) < hoff
                oct_ = _lshr(hv, 29)
                m = valid & (oct_ == _o)
                plsc.store_compressed(
                    dst.at[pl.ds(jnp.minimum(f, hcap - L), L)], hv, mask=m)
                return f + jnp.sum(m.astype(jnp.int32))

            fill = lax.fori_loop(0, ng, bgroup, fill)
        soff_sm[base_sm + 8] = fill

    bucketize(ehits, eoff, ECAP, ebuck, 0)
    bucketize(rhits, roff, RCAP, rbuck, 16)

    def fire(tbl, buf, ch):
        start = my_lo + ch * (CB * 128)
        pltpu.async_copy(tbl.at[:, pl.ds(start, CB * 128)], buf, sem)

    def drain(tbl, buf):
        pltpu.make_async_copy(
            tbl.at[:, pl.ds(0, CB * 128)], buf, sem).wait()

    def flush(fill):
        @pl.when(fill >= 64)
        def _():
            pltpu.sync_copy(stag, S.at[plsc.Indices(sidxa, ignored_value=-1)],
                            add=True)
            for z in range(4):
                sidxa[pl.ds(z * L, L)] = jnp.full((L,), -1, jnp.int32)

            def zr(i, carry):
                for k in range(8):
                    stag[i, pl.ds(k * L, L)] = jnp.zeros((L,), jnp.float32)
                return carry
            lax.fori_loop(0, 64, zr, 0)
        return jnp.where(fill >= 64, 0, fill)

    def append_rows(f, m, srow, half, getval):
        rows = f + lanes
        for d in range(D):
            val = getval(d)
            plsc.store_scatter(stag, [rows, half + d], val)
        sidxa[pl.ds(f, L)] = jnp.where(m, srow, -1)

    def dense_group(buf, hv, valid, is_rel, fill):
        f = flush(fill)
        vloc, pos = _unpack(hv)
        vloc = jnp.where(valid, vloc, 0)
        pos = jnp.where(valid, pos, 0)
        blk = _lshr(vloc, 7)
        il = vloc & 127
        edge, role = _edge_role(pos)
        if is_rel:
            sign = jnp.ones((L,), jnp.float32)
        else:
            sign = jnp.where(role == 2, -1.0, 1.0).astype(jnp.float32)
        srow = _lshr(edge, 1)
        half = (edge & 1) * 64
        colv = blk * 128 + il

        def getval(d):
            return plsc.load_gather(
                buf, [jnp.full((L,), d, jnp.int32), colv]) * sign

        append_rows(f, valid, srow, half, getval)
        return f + L

    def process_chunk(ch, buf, is_rel, fill):
        win_lo = ch * (CB * 128)
        base_sm = 16 if is_rel else 0
        oct_ = lax.div(ch, 16)
        soff = soff_sm[base_sm + oct_]
        send = soff_sm[base_sm + oct_ + 1]
        buck = rbuck if is_rel else ebuck

        def cgroup(i, cfill):
            hv = buck[pl.ds(soff + i * L, L)]
            valid = (soff + i * L + lanes) < send
            vloc = _lshr(hv, 17)
            inwin = valid & (vloc >= win_lo) & (vloc < win_lo + CB * 128)
            plsc.store_compressed(
                clist.at[pl.ds(jnp.minimum(cfill, CCAP - L), L)], hv,
                mask=inwin)
            return cfill + jnp.sum(inwin.astype(jnp.int32))

        ng = lax.div(send - soff + (L - 1), L)
        cfill = lax.fori_loop(0, ng, cgroup, jnp.int32(0))

        def dgroup(i, fill):
            hv = clist[pl.ds(i * L, L)]
            valid = (i * L + lanes) < cfill
            vloc, pos = _unpack(hv)
            hv_local = _lshl(vloc - win_lo, 17) | pos
            return dense_group(buf, hv_local, valid, is_rel, fill)

        ncg = lax.div(cfill + (L - 1), L)
        return lax.fori_loop(0, ncg, dgroup, fill)

    def sweep(tbl, is_rel, fill):
        fire(tbl, bufA, 0)

        def pair(g, fill):
            ch = 2 * g
            fire(tbl, bufB, ch + 1)
            drain(tbl, bufA)
            fill = process_chunk(ch, bufA, is_rel, fill)
            fire(tbl, bufA, ch + 2)
            drain(tbl, bufB)
            fill = process_chunk(ch + 1, bufB, is_rel, fill)
            return fill

        fill = lax.fori_loop(0, NCH // 2, pair, fill)
        drain(tbl, bufA)
        return fill

    fill = jnp.int32(0)
    fill = sweep(entT_ref, False, fill)
    fill = sweep(relT_ref, True, fill)

    def tail_do(buf, m, pos, eloc, sub_lo, is_rel, fill):
        f = flush(fill)

        @pl.when(jnp.any(m))
        def _():
            colv = jnp.where(m, eloc - sub_lo, 0)
            posx = jnp.where(m, pos, 0)
            edge, role2 = _edge_role(posx)
            if is_rel:
                sign = jnp.ones((L,), jnp.float32)
            else:
                sign = jnp.where(role2 == 2, -1.0, 1.0).astype(jnp.float32)
            srow = _lshr(edge, 1)
            half = (edge & 1) * 64

            def getval(d):
                return plsc.load_gather(
                    buf, [jnp.full((L,), d, jnp.int32), colv]) * sign

            append_rows(f, m, srow, half, getval)

        return jnp.where(jnp.any(m), f + L, f)

    def tail_pass(tail_ref, is_rel, fill):
        pltpu.sync_copy(tail_ref.at[:, pl.ds(0, 256)], bufA)
        pltpu.sync_copy(tail_ref.at[:, pl.ds(256, 256)], bufB)
        pltpu.sync_copy(edge_ref.at[pl.ds(wid * (POS // NW), POS // NW)],
                        slab.at[pl.ds(0, POS // NW)])

        def tprep(i):
            v = slab[pl.ds(i * L, L)]
            pos = wid * (POS // NW) + i * L + lanes
            _, role = _edge_role(pos)
            mrole = (role == 1) if is_rel else (role != 1)
            mt = mrole & (v >= MAIN)
            eloc = jnp.where(mt, v - MAIN, 0)
            return pos, mt, eloc

        def tgroup(i, fill):
            pos, mt, eloc = tprep(i)
            fill = tail_do(bufA, mt & (eloc < 256), pos, eloc, 0, is_rel, fill)
            fill = tail_do(bufB, mt & (eloc >= 256) & (eloc < 512), pos, eloc,
                           256, is_rel, fill)
            return fill

        fill = lax.fori_loop(0, (POS // NW) // L, tgroup, fill)

        pltpu.sync_copy(tail_ref.at[:, pl.ds(512, 256)], bufA)

        def tgroup2(i, fill):
            pos, mt, eloc = tprep(i)
            return tail_do(bufA, mt & (eloc >= 512), pos, eloc, 512, is_rel,
                           fill)

        return lax.fori_loop(0, (POS // NW) // L, tgroup2, fill)

    fill = tail_pass(etail_ref, False, fill)
    fill = tail_pass(rtail_ref, True, fill)

    @pl.when(fill > 0)
    def _():
        pltpu.sync_copy(stag, S.at[plsc.Indices(sidxa, ignored_value=-1)],
                        add=True)

    plsc.subcore_barrier()
    pltpu.sync_copy(S.at[pl.ds(s * 512, 512)],
                    out_ref.at[c, pl.ds(s * 512, 512), :])


def _merge_body(parts_ref, out_ref, p0, p1, partE, partO, out_v, sem):
    c = lax.axis_index("c")
    s = lax.axis_index("s")
    wid = s * NC + c
    r0 = wid * 256
    lanes = lax.iota(jnp.int32, L)
    pltpu.sync_copy(parts_ref.at[0, pl.ds(r0, 256), :], p0)
    pltpu.sync_copy(parts_ref.at[1, pl.ds(r0, 256), :], p1)

    def group(g, carry):
        for e in range(L):
            row = g * L + e
            accE = None
            accO = None
            for k in range(8):
                sl = pl.ds(k * L, L)
                v = jnp.abs(p0[row, sl] + p1[row, sl])
                if k < 4:
                    accE = v if accE is None else accE + v
                else:
                    accO = v if accO is None else accO + v
            partE[e, :] = accE
            partO[e, :] = accO
        totE = jnp.zeros((L,), jnp.float32)
        totO = jnp.zeros((L,), jnp.float32)
        for j in range(L):
            cj = jnp.full((L,), j, jnp.int32)
            totE = totE + plsc.load_gather(partE, [lanes, cj])
            totO = totO + plsc.load_gather(partO, [lanes, cj])
        base = g * 2 * L
        plsc.store_scatter(out_v, [base + 2 * lanes], totE)
        plsc.store_scatter(out_v, [base + 2 * lanes + 1], totO)
        return carry

    lax.fori_loop(0, 16, group, 0)
    pltpu.sync_copy(out_v, out_ref.at[pl.ds(wid * 512, 512)])


@functools.partial(jax.jit, static_argnames=())
def _transe_sc(edge_flat, entT, relT, etail, rtail):
    mesh = plsc.VectorSubcoreMesh(core_axis_name="c", subcore_axis_name="s")
    cp = pltpu.CompilerParams(
        needs_layout_passes=False, use_tc_tiling_on_sc=True)
    k1 = pl.kernel(
        _sweep_body,
        out_type=jax.ShapeDtypeStruct((2, 8192, 128), jnp.float32),
        mesh=mesh,
        compiler_params=cp,
        scratch_types=[
            pltpu.VMEM((64, CB * 128), jnp.float32),
            pltpu.VMEM((64, CB * 128), jnp.float32),
            pltpu.VMEM((SLAB,), jnp.int32),
            pltpu.VMEM((ECAP,), jnp.int32),
            pltpu.VMEM((RCAP,), jnp.int32),
            pltpu.VMEM((ECAP,), jnp.int32),
            pltpu.VMEM((RCAP,), jnp.int32),
            pltpu.VMEM((CCAP,), jnp.int32),
            pltpu.VMEM((64, 128), jnp.float32),
            pltpu.VMEM((64,), jnp.int32),
            pltpu.SMEM((32,), jnp.int32),
            pltpu.VMEM_SHARED((8192, 128), jnp.float32),
            pltpu.SemaphoreType.DMA,
        ],
    )
    parts = k1(edge_flat, entT, relT, etail, rtail)
    k2 = pl.kernel(
        _merge_body,
        out_type=jax.ShapeDtypeStruct((B,), jnp.float32),
        mesh=mesh,
        compiler_params=cp,
        scratch_types=[
            pltpu.VMEM((256, 128), jnp.float32),
            pltpu.VMEM((256, 128), jnp.float32),
            pltpu.VMEM((L, L), jnp.float32),
            pltpu.VMEM((L, L), jnp.float32),
            pltpu.VMEM((512,), jnp.float32),
            pltpu.SemaphoreType.DMA,
        ],
    )
    return k2(parts)


def kernel(edge, entity_embedding, relation_embedding):
    pad = ((0, 0), (0, 768 - TAIL))
    etail = jnp.pad(entity_embedding[MAIN:].T, pad)
    rtail = jnp.pad(relation_embedding[MAIN:].T, pad)
    return _transe_sc(edge.reshape(-1), entity_embedding.T,
                      relation_embedding.T, etail, rtail)


# speedup vs baseline: 1.4702x; 1.0668x over previous
import functools

import jax
import jax.numpy as jnp
from jax import lax
from jax.experimental import pallas as pl
from jax.experimental.pallas import tpu as pltpu
from jax.experimental.pallas import tpu_sc as plsc

NC = 2
NS = 16
L = 16
NW = NC * NS

B = 16384
POS = B * 3
D = 64
NE = 1_000_000

CB = 2
NBPW = 244
NCH = NBPW // CB
EPW = NBPW * 128
MAIN = NW * EPW
TAIL = NE - MAIN
ECAP = 6144
RCAP = 3072
CCAP = 1024
SLAB = 1536
MAGIC3 = 43691


def _lshr(x, k):
    u = lax.bitcast_convert_type(x, jnp.uint32)
    return lax.bitcast_convert_type(u >> jnp.uint32(k), jnp.int32)


def _lshl(x, k):
    u = lax.bitcast_convert_type(x, jnp.uint32)
    return lax.bitcast_convert_type(u << jnp.uint32(k), jnp.int32)


def _unpack(hv):
    return _lshr(hv, 17), hv & 0x1FFFF


def _edge_role(pos):
    edge = _lshr(pos * MAGIC3, 17)
    role = pos - edge * 3
    return edge, role


def _sweep_body(edge_ref, entT_ref, relT_ref, etail_ref, rtail_ref, out_ref,
                bufA, bufB, slab, ehits, rhits, ebuck, rbuck, clist,
                stag, sidxa, soff_sm, S, sem):
    c = lax.axis_index("c")
    s = lax.axis_index("s")
    wid = s * NC + c
    my_lo = wid * EPW
    lanes = lax.iota(jnp.int32, L)

    def zrow(i, carry):
        for k in range(8):
            stag[i, pl.ds(k * L, L)] = jnp.zeros((L,), jnp.float32)
        return carry
    lax.fori_loop(0, 64, zrow, 0)
    for z in range(8):
        pltpu.sync_copy(stag, S.at[pl.ds(s * 512 + z * 64, 64)])
    for z in range(4):
        sidxa[pl.ds(z * L, L)] = jnp.full((L,), -1, jnp.int32)
    plsc.subcore_barrier()

    def scan_group(i, carry):
        eoff, roff, p0 = carry
        v = slab[pl.ds(i * L, L)]
        pos = p0 + i * L + lanes
        _, role = _edge_role(pos)
        inr = (v >= my_lo) & (v < my_lo + EPW)
        packed = _lshl(v - my_lo, 17) | pos
        me = inr & (role != 1)
        mr = inr & (role == 1)
        plsc.store_compressed(
            ehits.at[pl.ds(jnp.minimum(eoff, ECAP - L), L)], packed, mask=me)
        plsc.store_compressed(
            rhits.at[pl.ds(jnp.minimum(roff, RCAP - L), L)], packed, mask=mr)
        return (eoff + jnp.sum(me.astype(jnp.int32)),
                roff + jnp.sum(mr.astype(jnp.int32)), p0)

    eoff = jnp.int32(0)
    roff = jnp.int32(0)
    for sl in range(POS // SLAB):
        pltpu.sync_copy(edge_ref.at[pl.ds(sl * SLAB, SLAB)], slab)
        eoff, roff, _ = lax.fori_loop(
            0, SLAB // L, scan_group, (eoff, roff, jnp.int32(sl * SLAB)))

    def bucketize(hits, hoff, hcap, dst, base_sm):
        ng = lax.div(hoff + (L - 1), L)
        fill = jnp.int32(0)
        for o in range(8):
            soff_sm[base_sm + o] = fill

            def bgroup(i, f, _o=o):
                hv = hits[pl.ds(i * L, L)]
                valid = (i * L + lanes) < hoff
                oct_ = _lshr(hv, 29)
                m = valid & (oct_ == _o)
                plsc.store_compressed(
                    dst.at[pl.ds(jnp.minimum(f, hcap - L), L)], hv, mask=m)
                return f + jnp.sum(m.astype(jnp.int32))

            fill = lax.fori_loop(0, ng, bgroup, fill)
        soff_sm[base_sm + 8] = fill

    bucketize(ehits, eoff, ECAP, ebuck, 0)
    bucketize(rhits, roff, RCAP, rbuck, 16)

    def fire(tbl, buf, ch):
        start = my_lo + ch * (CB * 128)
        pltpu.async_copy(tbl.at[:, pl.ds(start, CB * 128)], buf, sem)

    def drain(tbl, buf):
        pltpu.make_async_copy(
            tbl.at[:, pl.ds(0, CB * 128)], buf, sem).wait()

    def flush(fill):
        @pl.when(fill >= 64)
        def _():
            pltpu.sync_copy(stag, S.at[plsc.Indices(sidxa, ignored_value=-1)],
                            add=True)
            for z in range(4):
                sidxa[pl.ds(z * L, L)] = jnp.full((L,), -1, jnp.int32)

            def zr(i, carry):
                for k in range(8):
                    stag[i, pl.ds(k * L, L)] = jnp.zeros((L,), jnp.float32)
                return carry
            lax.fori_loop(0, 64, zr, 0)
        return jnp.where(fill >= 64, 0, fill)

    def append_rows(f, m, srow, half, getval):
        rows = f + lanes
        rot = srow & 127
        for d in range(D):
            val = getval(d)
            plsc.store_scatter(stag, [rows, (half + d + rot) & 127], val)
        sidxa[pl.ds(f, L)] = jnp.where(m, srow, -1)

    def dense_group(buf, hv, valid, is_rel, fill):
        f = flush(fill)
        vloc, pos = _unpack(hv)
        vloc = jnp.where(valid, vloc, 0)
        pos = jnp.where(valid, pos, 0)
        blk = _lshr(vloc, 7)
        il = vloc & 127
        edge, role = _edge_role(pos)
        if is_rel:
            sign = jnp.ones((L,), jnp.float32)
        else:
            sign = jnp.where(role == 2, -1.0, 1.0).astype(jnp.float32)
        srow = _lshr(edge, 1)
        half = (edge & 1) * 64
        colv = blk * 128 + il

        def getval(d):
            return plsc.load_gather(
                buf, [jnp.full((L,), d, jnp.int32), colv]) * sign

        append_rows(f, valid, srow, half, getval)
        return f + L

    def process_chunk(ch, buf, is_rel, fill):
        win_lo = ch * (CB * 128)
        base_sm = 16 if is_rel else 0
        oct_ = lax.div(ch, 16)
        soff = soff_sm[base_sm + oct_]
        send = soff_sm[base_sm + oct_ + 1]
        buck = rbuck if is_rel else ebuck

        def cgroup(i, cfill):
            hv = buck[pl.ds(soff + i * L, L)]
            valid = (soff + i * L + lanes) < send
            vloc = _lshr(hv, 17)
            inwin = valid & (vloc >= win_lo) & (vloc < win_lo + CB * 128)
            plsc.store_compressed(
                clist.at[pl.ds(jnp.minimum(cfill, CCAP - L), L)], hv,
                mask=inwin)
            return cfill + jnp.sum(inwin.astype(jnp.int32))

        ng = lax.div(send - soff + (L - 1), L)
        cfill = lax.fori_loop(0, ng, cgroup, jnp.int32(0))

        def dgroup(i, fill):
            hv = clist[pl.ds(i * L, L)]
            valid = (i * L + lanes) < cfill
            vloc, pos = _unpack(hv)
            hv_local = _lshl(vloc - win_lo, 17) | pos
            return dense_group(buf, hv_local, valid, is_rel, fill)

        ncg = lax.div(cfill + (L - 1), L)
        return lax.fori_loop(0, ncg, dgroup, fill)

    def sweep(tbl, is_rel, fill):
        fire(tbl, bufA, 0)

        def pair(g, fill):
            ch = 2 * g
            fire(tbl, bufB, ch + 1)
            drain(tbl, bufA)
            fill = process_chunk(ch, bufA, is_rel, fill)
            fire(tbl, bufA, ch + 2)
            drain(tbl, bufB)
            fill = process_chunk(ch + 1, bufB, is_rel, fill)
            return fill

        fill = lax.fori_loop(0, NCH // 2, pair, fill)
        drain(tbl, bufA)
        return fill

    fill = jnp.int32(0)
    fill = sweep(entT_ref, False, fill)
    fill = sweep(relT_ref, True, fill)

    def tail_do(buf, m, pos, eloc, sub_lo, is_rel, fill):
        f = flush(fill)

        @pl.when(jnp.any(m))
        def _():
            colv = jnp.where(m, eloc - sub_lo, 0)
            posx = jnp.where(m, pos, 0)
            edge, role2 = _edge_role(posx)
            if is_rel:
                sign = jnp.ones((L,), jnp.float32)
            else:
                sign = jnp.where(role2 == 2, -1.0, 1.0).astype(jnp.float32)
            srow = _lshr(edge, 1)
            half = (edge & 1) * 64

            def getval(d):
                return plsc.load_gather(
                    buf, [jnp.full((L,), d, jnp.int32), colv]) * sign

            append_rows(f, m, srow, half, getval)

        return jnp.where(jnp.any(m), f + L, f)

    def tail_pass(tail_ref, is_rel, fill):
        pltpu.sync_copy(tail_ref.at[:, pl.ds(0, 256)], bufA)
        pltpu.sync_copy(tail_ref.at[:, pl.ds(256, 256)], bufB)
        pltpu.sync_copy(edge_ref.at[pl.ds(wid * (POS // NW), POS // NW)],
                        slab.at[pl.ds(0, POS // NW)])

        def tprep(i):
            v = slab[pl.ds(i * L, L)]
            pos = wid * (POS // NW) + i * L + lanes
            _, role = _edge_role(pos)
            mrole = (role == 1) if is_rel else (role != 1)
            mt = mrole & (v >= MAIN)
            eloc = jnp.where(mt, v - MAIN, 0)
            return pos, mt, eloc

        def tgroup(i, fill):
            pos, mt, eloc = tprep(i)
            fill = tail_do(bufA, mt & (eloc < 256), pos, eloc, 0, is_rel, fill)
            fill = tail_do(bufB, mt & (eloc >= 256) & (eloc < 512), pos, eloc,
                           256, is_rel, fill)
            return fill

        fill = lax.fori_loop(0, (POS // NW) // L, tgroup, fill)

        pltpu.sync_copy(tail_ref.at[:, pl.ds(512, 256)], bufA)

        def tgroup2(i, fill):
            pos, mt, eloc = tprep(i)
            return tail_do(bufA, mt & (eloc >= 512), pos, eloc, 512, is_rel,
                           fill)

        return lax.fori_loop(0, (POS // NW) // L, tgroup2, fill)

    fill = tail_pass(etail_ref, False, fill)
    fill = tail_pass(rtail_ref, True, fill)

    @pl.when(fill > 0)
    def _():
        pltpu.sync_copy(stag, S.at[plsc.Indices(sidxa, ignored_value=-1)],
                        add=True)

    plsc.subcore_barrier()
    pltpu.sync_copy(S.at[pl.ds(s * 512, 512)],
                    out_ref.at[c, pl.ds(s * 512, 512), :])


def _merge_body(parts_ref, out_ref, p0, p1, partE, partO, out_v, sem):
    c = lax.axis_index("c")
    s = lax.axis_index("s")
    wid = s * NC + c
    r0 = wid * 256
    lanes = lax.iota(jnp.int32, L)
    pltpu.sync_copy(parts_ref.at[0, pl.ds(r0, 256), :], p0)
    pltpu.sync_copy(parts_ref.at[1, pl.ds(r0, 256), :], p1)

    def group(g, carry):
        for e in range(L):
            row = g * L + e
            rot = (r0 + row) & 127
            rowv = jnp.full((L,), row, jnp.int32)
            accE = None
            accO = None
            for k in range(8):
                colr = (lanes + (k * L + rot)) & 127
                v = jnp.abs(plsc.load_gather(p0, [rowv, colr])
                            + plsc.load_gather(p1, [rowv, colr]))
                if k < 4:
                    accE = v if accE is None else accE + v
                else:
                    accO = v if accO is None else accO + v
            partE[e, :] = accE
            partO[e, :] = accO
        totE = jnp.zeros((L,), jnp.float32)
        totO = jnp.zeros((L,), jnp.float32)
        for j in range(L):
            cj = jnp.full((L,), j, jnp.int32)
            totE = totE + plsc.load_gather(partE, [lanes, cj])
            totO = totO + plsc.load_gather(partO, [lanes, cj])
        base = g * 2 * L
        plsc.store_scatter(out_v, [base + 2 * lanes], totE)
        plsc.store_scatter(out_v, [base + 2 * lanes + 1], totO)
        return carry

    lax.fori_loop(0, 16, group, 0)
    pltpu.sync_copy(out_v, out_ref.at[pl.ds(wid * 512, 512)])


@functools.partial(jax.jit, static_argnames=())
def _transe_sc(edge_flat, entT, relT, etail, rtail):
    mesh = plsc.VectorSubcoreMesh(core_axis_name="c", subcore_axis_name="s")
    cp = pltpu.CompilerParams(
        needs_layout_passes=False, use_tc_tiling_on_sc=True)
    k1 = pl.kernel(
        _sweep_body,
        out_type=jax.ShapeDtypeStruct((2, 8192, 128), jnp.float32),
        mesh=mesh,
        compiler_params=cp,
        scratch_types=[
            pltpu.VMEM((64, CB * 128), jnp.float32),
            pltpu.VMEM((64, CB * 128), jnp.float32),
            pltpu.VMEM((SLAB,), jnp.int32),
            pltpu.VMEM((ECAP,), jnp.int32),
            pltpu.VMEM((RCAP,), jnp.int32),
            pltpu.VMEM((ECAP,), jnp.int32),
            pltpu.VMEM((RCAP,), jnp.int32),
            pltpu.VMEM((CCAP,), jnp.int32),
            pltpu.VMEM((64, 128), jnp.float32),
            pltpu.VMEM((64,), jnp.int32),
            pltpu.SMEM((32,), jnp.int32),
            pltpu.VMEM_SHARED((8192, 128), jnp.float32),
            pltpu.SemaphoreType.DMA,
        ],
    )
    parts = k1(edge_flat, entT, relT, etail, rtail)
    k2 = pl.kernel(
        _merge_body,
        out_type=jax.ShapeDtypeStruct((B,), jnp.float32),
        mesh=mesh,
        compiler_params=cp,
        scratch_types=[
            pltpu.VMEM((256, 128), jnp.float32),
            pltpu.VMEM((256, 128), jnp.float32),
            pltpu.VMEM((L, L), jnp.float32),
            pltpu.VMEM((L, L), jnp.float32),
            pltpu.VMEM((512,), jnp.float32),
            pltpu.SemaphoreType.DMA,
        ],
    )
    return k2(parts)


def kernel(edge, entity_embedding, relation_embedding):
    pad = ((0, 0), (0, 768 - TAIL))
    etail = jnp.pad(entity_embedding[MAIN:].T, pad)
    rtail = jnp.pad(relation_embedding[MAIN:].T, pad)
    return _transe_sc(edge.reshape(-1), entity_embedding.T,
                      relation_embedding.T, etail, rtail)


# speedup vs baseline: 1.7793x; 1.2103x over previous
import functools

import jax
import jax.numpy as jnp
from jax import lax
from jax.experimental import pallas as pl
from jax.experimental.pallas import tpu as pltpu
from jax.experimental.pallas import tpu_sc as plsc

NC = 2
NS = 16
L = 16
NW = NC * NS

B = 16384
POS = B * 3
D = 64
NE = 1_000_000

CB = 2
NBPW = 244
NCH = NBPW // CB
EPW = NBPW * 128
MAIN = NW * EPW
TAIL = NE - MAIN
ECAP = 3072
RCAP = 1536
CCAP = 1024
SLAB = 1536
MAGIC3 = 43691


def _lshr(x, k):
    u = lax.bitcast_convert_type(x, jnp.uint32)
    return lax.bitcast_convert_type(u >> jnp.uint32(k), jnp.int32)


def _lshl(x, k):
    u = lax.bitcast_convert_type(x, jnp.uint32)
    return lax.bitcast_convert_type(u << jnp.uint32(k), jnp.int32)


def _unpack(hv):
    return _lshr(hv, 17), hv & 0x1FFFF


def _edge_role(pos):
    edge = _lshr(pos * MAGIC3, 17)
    role = pos - edge * 3
    return edge, role


def _sweep_body(edge_ref, entT_ref, relT_ref, etail_ref, rtail_ref, out_ref,
                bufA, bufB, bufC, slab, ehits, rhits, ebuck, rbuck, clist,
                stag, sidxa, soff_sm, S, sem):
    c = lax.axis_index("c")
    s = lax.axis_index("s")
    wid = s * NC + c
    my_lo = wid * EPW
    lanes = lax.iota(jnp.int32, L)

    def zrow(i, carry):
        for k in range(8):
            stag[i, pl.ds(k * L, L)] = jnp.zeros((L,), jnp.float32)
        return carry
    lax.fori_loop(0, 32, zrow, 0)
    for z in range(16):
        pltpu.sync_copy(stag, S.at[pl.ds(s * 512 + z * 32, 32)])
    for z in range(2):
        sidxa[pl.ds(z * L, L)] = jnp.full((L,), -1, jnp.int32)
    plsc.subcore_barrier()

    def scan_group(i, carry):
        eoff, roff, p0 = carry
        v = slab[pl.ds(i * L, L)]
        pos = p0 + i * L + lanes
        _, role = _edge_role(pos)
        inr = (v >= my_lo) & (v < my_lo + EPW)
        packed = _lshl(v - my_lo, 17) | pos
        me = inr & (role != 1)
        mr = inr & (role == 1)
        plsc.store_compressed(
            ehits.at[pl.ds(jnp.minimum(eoff, ECAP - L), L)], packed, mask=me)
        plsc.store_compressed(
            rhits.at[pl.ds(jnp.minimum(roff, RCAP - L), L)], packed, mask=mr)
        return (eoff + jnp.sum(me.astype(jnp.int32)),
                roff + jnp.sum(mr.astype(jnp.int32)), p0)

    eoff = jnp.int32(0)
    roff = jnp.int32(0)
    for sl in range(POS // SLAB):
        pltpu.sync_copy(edge_ref.at[pl.ds(sl * SLAB, SLAB)], slab)
        eoff, roff, _ = lax.fori_loop(
            0, SLAB // L, scan_group, (eoff, roff, jnp.int32(sl * SLAB)))

    def bucketize(hits, hoff, hcap, dst, base_sm):
        ng = lax.div(hoff + (L - 1), L)
        fill = jnp.int32(0)
        for o in range(8):
            soff_sm[base_sm + o] = fill

            def bgroup(i, f, _o=o):
                hv = hits[pl.ds(i * L, L)]
                valid = (i * L + lanes) < hoff
                oct_ = _lshr(hv, 29)
                m = valid & (oct_ == _o)
                plsc.store_compressed(
                    dst.at[pl.ds(jnp.minimum(f, hcap - L), L)], hv, mask=m)
                return f + jnp.sum(m.astype(jnp.int32))

            fill = lax.fori_loop(0, ng, bgroup, fill)
        soff_sm[base_sm + 8] = fill

    bucketize(ehits, eoff, ECAP, ebuck, 0)
    bucketize(rhits, roff, RCAP, rbuck, 16)

    def fire(tbl, buf, ch):
        start = my_lo + ch * (CB * 128)
        pltpu.async_copy(tbl.at[:, pl.ds(start, CB * 128)], buf, sem)

    def drain(tbl, buf):
        pltpu.make_async_copy(
            tbl.at[:, pl.ds(0, CB * 128)], buf, sem).wait()

    def flush(fill):
        @pl.when(fill >= 32)
        def _():
            pltpu.sync_copy(stag, S.at[plsc.Indices(sidxa, ignored_value=-1)],
                            add=True)
            for z in range(2):
                sidxa[pl.ds(z * L, L)] = jnp.full((L,), -1, jnp.int32)

            def zr(i, carry):
                for k in range(8):
                    stag[i, pl.ds(k * L, L)] = jnp.zeros((L,), jnp.float32)
                return carry
            lax.fori_loop(0, 32, zr, 0)
        return jnp.where(fill >= 32, 0, fill)

    def append_rows(f, m, srow, half, getval):
        rows = f + lanes
        rot = srow & 127
        def dblk(j, carry):
            for k in range(8):
                d = j * 8 + k
                val = getval(d)
                plsc.store_scatter(stag, [rows, (half + d + rot) & 127], val)
            return carry
        lax.fori_loop(0, D // 8, dblk, 0)
        sidxa[pl.ds(f, L)] = jnp.where(m, srow, -1)

    def dense_group(buf, hv, valid, is_rel, fill):
        f = flush(fill)
        vloc, pos = _unpack(hv)
        vloc = jnp.where(valid, vloc, 0)
        pos = jnp.where(valid, pos, 0)
        blk = _lshr(vloc, 7)
        il = vloc & 127
        edge, role = _edge_role(pos)
        if is_rel:
            sign = jnp.ones((L,), jnp.float32)
        else:
            sign = jnp.where(role == 2, -1.0, 1.0).astype(jnp.float32)
        srow = _lshr(edge, 1)
        half = (edge & 1) * 64
        colv = blk * 128 + il

        def getval(d):
            dv = jnp.full((L,), 1, jnp.int32) * d
            return plsc.load_gather(buf, [dv, colv]) * sign

        append_rows(f, valid, srow, half, getval)
        return f + L

    def process_chunk(ch, buf, is_rel, fill):
        win_lo = ch * (CB * 128)
        base_sm = 16 if is_rel else 0
        oct_ = lax.div(ch, 16)
        soff = soff_sm[base_sm + oct_]
        send = soff_sm[base_sm + oct_ + 1]
        buck = rbuck if is_rel else ebuck

        def cgroup(i, cfill):
            hv = buck[pl.ds(soff + i * L, L)]
            valid = (soff + i * L + lanes) < send
            vloc = _lshr(hv, 17)
            inwin = valid & (vloc >= win_lo) & (vloc < win_lo + CB * 128)
            plsc.store_compressed(
                clist.at[pl.ds(jnp.minimum(cfill, CCAP - L), L)], hv,
                mask=inwin)
            return cfill + jnp.sum(inwin.astype(jnp.int32))

        ng = lax.div(send - soff + (L - 1), L)
        cfill = lax.fori_loop(0, ng, cgroup, jnp.int32(0))

        def dgroup(i, fill):
            hv = clist[pl.ds(i * L, L)]
            valid = (i * L + lanes) < cfill
            vloc, pos = _unpack(hv)
            hv_local = _lshl(vloc - win_lo, 17) | pos
            return dense_group(buf, hv_local, valid, is_rel, fill)

        ncg = lax.div(cfill + (L - 1), L)
        return lax.fori_loop(0, ncg, dgroup, fill)

    def sweep(tbl, is_rel, fill):
        fire(tbl, bufA, 0)
        fire(tbl, bufB, 1)

        def trip(g, fill):
            ch = 3 * g
            fire(tbl, bufC, ch + 2)
            drain(tbl, bufA)
            fill = process_chunk(ch, bufA, is_rel, fill)
            fire(tbl, bufA, ch + 3)
            drain(tbl, bufB)
            fill = process_chunk(ch + 1, bufB, is_rel, fill)
            fire(tbl, bufB, ch + 4)
            drain(tbl, bufC)
            fill = process_chunk(ch + 2, bufC, is_rel, fill)
            return fill

        fill = lax.fori_loop(0, NCH // 3, trip, fill)
        drain(tbl, bufA)
        fill = process_chunk(NCH - 2, bufA, is_rel, fill)
        drain(tbl, bufB)
        fill = process_chunk(NCH - 1, bufB, is_rel, fill)
        return fill

    fill = jnp.int32(0)
    fill = sweep(entT_ref, False, fill)
    fill = sweep(relT_ref, True, fill)

    def tail_do(buf, m, pos, eloc, sub_lo, is_rel, fill):
        f = flush(fill)

        @pl.when(jnp.any(m))
        def _():
            colv = jnp.where(m, eloc - sub_lo, 0)
            posx = jnp.where(m, pos, 0)
            edge, role2 = _edge_role(posx)
            if is_rel:
                sign = jnp.ones((L,), jnp.float32)
            else:
                sign = jnp.where(role2 == 2, -1.0, 1.0).astype(jnp.float32)
            srow = _lshr(edge, 1)
            half = (edge & 1) * 64

            def getval(d):
                dv = jnp.full((L,), 1, jnp.int32) * d
                return plsc.load_gather(buf, [dv, colv]) * sign

            append_rows(f, m, srow, half, getval)

        return jnp.where(jnp.any(m), f + L, f)

    def tail_pass(tail_ref, is_rel, fill):
        pltpu.sync_copy(tail_ref.at[:, pl.ds(0, 256)], bufA)
        pltpu.sync_copy(tail_ref.at[:, pl.ds(256, 256)], bufB)
        pltpu.sync_copy(edge_ref.at[pl.ds(wid * (POS // NW), POS // NW)],
                        slab.at[pl.ds(0, POS // NW)])

        def tprep(i):
            v = slab[pl.ds(i * L, L)]
            pos = wid * (POS // NW) + i * L + lanes
            _, role = _edge_role(pos)
            mrole = (role == 1) if is_rel else (role != 1)
            mt = mrole & (v >= MAIN)
            eloc = jnp.where(mt, v - MAIN, 0)
            return pos, mt, eloc

        def tgroup(i, fill):
            pos, mt, eloc = tprep(i)
            fill = tail_do(bufA, mt & (eloc < 256), pos, eloc, 0, is_rel, fill)
            fill = tail_do(bufB, mt & (eloc >= 256) & (eloc < 512), pos, eloc,
                           256, is_rel, fill)
            return fill

        fill = lax.fori_loop(0, (POS // NW) // L, tgroup, fill)

        pltpu.sync_copy(tail_ref.at[:, pl.ds(512, 256)], bufA)

        def tgroup2(i, fill):
            pos, mt, eloc = tprep(i)
            return tail_do(bufA, mt & (eloc >= 512), pos, eloc, 512, is_rel,
                           fill)

        return lax.fori_loop(0, (POS // NW) // L, tgroup2, fill)

    fill = tail_pass(etail_ref, False, fill)
    fill = tail_pass(rtail_ref, True, fill)

    @pl.when(fill > 0)
    def _():
        pltpu.sync_copy(stag, S.at[plsc.Indices(sidxa, ignored_value=-1)],
                        add=True)

    plsc.subcore_barrier()
    pltpu.sync_copy(S.at[pl.ds(s * 512, 512)],
                    out_ref.at[c, pl.ds(s * 512, 512), :])


def _merge_body(parts_ref, out_ref, p0, p1, partE, partO, out_v, sem):
    c = lax.axis_index("c")
    s = lax.axis_index("s")
    wid = s * NC + c
    r0 = wid * 256
    lanes = lax.iota(jnp.int32, L)
    pltpu.sync_copy(parts_ref.at[0, pl.ds(r0, 256), :], p0)
    pltpu.sync_copy(parts_ref.at[1, pl.ds(r0, 256), :], p1)

    def group(g, carry):
        for e in range(L):
            row = g * L + e
            rot = (r0 + row) & 127
            rowv = jnp.full((L,), row, jnp.int32)
            accE = None
            accO = None
            for k in range(8):
                colr = (lanes + (k * L + rot)) & 127
                v = jnp.abs(plsc.load_gather(p0, [rowv, colr])
                            + plsc.load_gather(p1, [rowv, colr]))
                if k < 4:
                    accE = v if accE is None else accE + v
                else:
                    accO = v if accO is None else accO + v
            partE[e, :] = accE
            partO[e, :] = accO
        totE = jnp.zeros((L,), jnp.float32)
        totO = jnp.zeros((L,), jnp.float32)
        for j in range(L):
            cj = jnp.full((L,), j, jnp.int32)
            totE = totE + plsc.load_gather(partE, [lanes, cj])
            totO = totO + plsc.load_gather(partO, [lanes, cj])
        base = g * 2 * L
        plsc.store_scatter(out_v, [base + 2 * lanes], totE)
        plsc.store_scatter(out_v, [base + 2 * lanes + 1], totO)
        return carry

    lax.fori_loop(0, 16, group, 0)
    pltpu.sync_copy(out_v, out_ref.at[pl.ds(wid * 512, 512)])


@functools.partial(jax.jit, static_argnames=())
def _transe_sc(edge_flat, entT, relT, etail, rtail):
    mesh = plsc.VectorSubcoreMesh(core_axis_name="c", subcore_axis_name="s")
    cp = pltpu.CompilerParams(
        needs_layout_passes=False, use_tc_tiling_on_sc=True)
    k1 = pl.kernel(
        _sweep_body,
        out_type=jax.ShapeDtypeStruct((2, 8192, 128), jnp.float32),
        mesh=mesh,
        compiler_params=cp,
        scratch_types=[
            pltpu.VMEM((64, CB * 128), jnp.float32),
            pltpu.VMEM((64, CB * 128), jnp.float32),
            pltpu.VMEM((64, CB * 128), jnp.float32),
            pltpu.VMEM((SLAB,), jnp.int32),
            pltpu.VMEM((ECAP,), jnp.int32),
            pltpu.VMEM((RCAP,), jnp.int32),
            pltpu.VMEM((ECAP,), jnp.int32),
            pltpu.VMEM((RCAP,), jnp.int32),
            pltpu.VMEM((CCAP,), jnp.int32),
            pltpu.VMEM((32, 128), jnp.float32),
            pltpu.VMEM((32,), jnp.int32),
            pltpu.SMEM((32,), jnp.int32),
            pltpu.VMEM_SHARED((8192, 128), jnp.float32),
            pltpu.SemaphoreType.DMA,
        ],
    )
    parts = k1(edge_flat, entT, relT, etail, rtail)
    k2 = pl.kernel(
        _merge_body,
        out_type=jax.ShapeDtypeStruct((B,), jnp.float32),
        mesh=mesh,
        compiler_params=cp,
        scratch_types=[
            pltpu.VMEM((256, 128), jnp.float32),
            pltpu.VMEM((256, 128), jnp.float32),
            pltpu.VMEM((L, L), jnp.float32),
            pltpu.VMEM((L, L), jnp.float32),
            pltpu.VMEM((512,), jnp.float32),
            pltpu.SemaphoreType.DMA,
        ],
    )
    return k2(parts)


def kernel(edge, entity_embedding, relation_embedding):
    pad = ((0, 0), (0, 768 - TAIL))
    etail = jnp.pad(entity_embedding[MAIN:].T, pad)
    rtail = jnp.pad(relation_embedding[MAIN:].T, pad)
    return _transe_sc(edge.reshape(-1), entity_embedding.T,
                      relation_embedding.T, etail, rtail)
